# Initial kernel scaffold; baseline (speedup 1.0000x reference)
import jax
import jax.numpy as jnp
from jax.experimental import pallas as pl


def kernel(x_t, probs, embedding_weight, omega_s, omega_a, omega_b):
    raise NotImplementedError("write your pallas kernel here")


# speedup vs baseline: 42.4536x; 42.4536x over previous
import functools

import jax
import jax.numpy as jnp
from jax.experimental import pallas as pl
from jax.experimental.pallas import tpu as pltpu

MASK_ID = 0
K = 8
T = 8


def _stats_kernel(p_ref, vals_ref, idx_ref, ent_ref, *, V):
    p = p_ref[...]
    col = jax.lax.broadcasted_iota(jnp.int32, (T, V), 1)
    safe = jnp.where(p > 0.0, p, 1.0)
    ent = jnp.sum(jnp.where(p > 0.0, -p * jnp.log(safe), 0.0), axis=1)
    ent_ref[0, 0, :] = ent
    pz = jnp.where(col == MASK_ID, -1.0, p)
    vals = []
    idxs = []
    for _ in range(K):
        m = jnp.max(pz, axis=1)
        am = jnp.min(jnp.where(pz == m[:, None], col, V), axis=1)
        vals.append(m)
        idxs.append(am)
        pz = jnp.where(col == am[:, None], -1.0, pz)
    vals_ref[0, :, :] = jnp.stack(vals, axis=1)
    idx_ref[0, :, :] = jnp.stack(idxs, axis=1)


def _combine_kernel(idx_sref, par_sref, real_ref, mask_ref, *rest):
    e_refs = rest[:K]
    out_ref = rest[K]
    t = pl.program_id(0)
    lam = par_sref[t * (2 + K) + 1]
    is_mask = par_sref[t * (2 + K)] > 0.5
    acc = par_sref[t * (2 + K) + 2] * e_refs[0][0, 0, :]
    for k in range(1, K):
        acc = acc + par_sref[t * (2 + K) + 2 + k] * e_refs[k][0, 0, :]
    soft = (1.0 - lam) * mask_ref[0, 0, :] + lam * acc
    out_ref[0, 0, :] = jnp.where(is_mask, soft, real_ref[0, 0, :])


def kernel(x_t, probs, embedding_weight, omega_s, omega_a, omega_b):
    B, S, V = probs.shape
    H = embedding_weight.shape[1]
    N = B * S
    NB = N // T
    p2 = probs.reshape(N, V).astype(jnp.float32)
    xt = x_t.reshape(N).astype(jnp.int32)

    vals, idx, ent = pl.pallas_call(
        functools.partial(_stats_kernel, V=V),
        grid=(NB,),
        in_specs=[pl.BlockSpec((T, V), lambda i: (i, 0))],
        out_specs=[
            pl.BlockSpec((1, T, K), lambda i: (i, 0, 0)),
            pl.BlockSpec((1, T, K), lambda i: (i, 0, 0)),
            pl.BlockSpec((1, 1, T), lambda i: (i, 0, 0)),
        ],
        out_shape=[
            jax.ShapeDtypeStruct((NB, T, K), jnp.float32),
            jax.ShapeDtypeStruct((NB, T, K), jnp.int32),
            jax.ShapeDtypeStruct((NB, 1, T), jnp.float32),
        ],
    )(p2)

    vals = vals.reshape(N, K)
    idx = idx.reshape(N, K)
    ent = ent.reshape(N)

    w = vals / (jnp.sum(vals, axis=1, keepdims=True) + 1e-10)
    r_s = jnp.clip(omega_s, 0.0, 1.0)
    r_a = jax.nn.softplus(omega_a)
    r_b = jax.nn.softplus(omega_b)
    lam = r_s * jax.nn.sigmoid(r_a * (r_b - ent))
    is_mask = (xt == MASK_ID).astype(jnp.float32)

    s_idx = jnp.concatenate([xt[:, None], idx], axis=1).reshape(-1)
    s_par = jnp.concatenate(
        [is_mask[:, None], lam[:, None], w], axis=1).reshape(-1)

    emb3 = embedding_weight.astype(jnp.float32).reshape(V, 1, H)

    def row_spec(fn):
        return pl.BlockSpec((1, 1, H), fn)

    in_specs = [
        row_spec(lambda t, si, sp: (si[t * (1 + K)], 0, 0)),
        row_spec(lambda t, si, sp: (MASK_ID, 0, 0)),
    ]
    for k in range(K):
        in_specs.append(
            row_spec(functools.partial(
                lambda t, si, sp, kk: (si[t * (1 + K) + 1 + kk], 0, 0), kk=k))
        )

    out = pl.pallas_call(
        _combine_kernel,
        grid_spec=pltpu.PrefetchScalarGridSpec(
            num_scalar_prefetch=2,
            grid=(N,),
            in_specs=in_specs,
            out_specs=pl.BlockSpec((1, 1, H), lambda t, si, sp: (t, 0, 0)),
        ),
        out_shape=jax.ShapeDtypeStruct((N, 1, H), jnp.float32),
    )(s_idx, s_par, *([emb3] * (2 + K)))

    return out.reshape(B, S, H)


# speedup vs baseline: 129.0969x; 3.0409x over previous
import functools

import jax
import jax.numpy as jnp
from jax.experimental import pallas as pl
from jax.experimental.pallas import tpu as pltpu

MASK_ID = 0
K = 8
T = 8


def _stats_kernel(flag_ref, p_ref, vals_ref, idx_ref, ent_ref, *, V):
    i = pl.program_id(0)

    @pl.when(flag_ref[i] != 0)
    def _compute():
        p = p_ref[...]
        col = jax.lax.broadcasted_iota(jnp.int32, (T, V), 1)
        safe = jnp.where(p > 0.0, p, 1.0)
        ent = jnp.sum(jnp.where(p > 0.0, -p * jnp.log(safe), 0.0), axis=1)
        ent_ref[0, 0, :] = ent
        pz = jnp.where(col == MASK_ID, -1.0, p)
        vals = []
        idxs = []
        for _ in range(K):
            m = jnp.max(pz, axis=1)
            am = jnp.min(jnp.where(pz == m[:, None], col, V), axis=1)
            vals.append(m)
            idxs.append(am)
            pz = jnp.where(col == am[:, None], -1.0, pz)
        vals_ref[0, :, :] = jnp.stack(vals, axis=1)
        idx_ref[0, :, :] = jnp.stack(idxs, axis=1)

    @pl.when(flag_ref[i] == 0)
    def _skip():
        vals_ref[...] = jnp.zeros_like(vals_ref)
        idx_ref[...] = jnp.zeros_like(idx_ref)
        ent_ref[...] = jnp.zeros_like(ent_ref)


def _combine_kernel(idx_sref, par_sref, real_ref, mask_ref, *rest):
    e_refs = rest[:K]
    out_ref = rest[K]
    t = pl.program_id(0)
    lam = par_sref[t * (2 + K) + 1]
    is_mask = par_sref[t * (2 + K)] > 0.5
    acc = par_sref[t * (2 + K) + 2] * e_refs[0][0, 0, :]
    for k in range(1, K):
        acc = acc + par_sref[t * (2 + K) + 2 + k] * e_refs[k][0, 0, :]
    soft = (1.0 - lam) * mask_ref[0, 0, :] + lam * acc
    out_ref[0, 0, :] = jnp.where(is_mask, soft, real_ref[0, 0, :])


def kernel(x_t, probs, embedding_weight, omega_s, omega_a, omega_b):
    B, S, V = probs.shape
    H = embedding_weight.shape[1]
    N = B * S
    NB = N // T
    p2 = probs.reshape(N, V).astype(jnp.float32)
    xt = x_t.reshape(N).astype(jnp.int32)

    blk_flags = jnp.any(xt.reshape(NB, T) == MASK_ID, axis=1).astype(jnp.int32)

    vals, idx, ent = pl.pallas_call(
        functools.partial(_stats_kernel, V=V),
        grid_spec=pltpu.PrefetchScalarGridSpec(
            num_scalar_prefetch=1,
            grid=(NB,),
            in_specs=[pl.BlockSpec((T, V), lambda i, f: (i, 0))],
            out_specs=[
                pl.BlockSpec((1, T, K), lambda i, f: (i, 0, 0)),
                pl.BlockSpec((1, T, K), lambda i, f: (i, 0, 0)),
                pl.BlockSpec((1, 1, T), lambda i, f: (i, 0, 0)),
            ],
        ),
        out_shape=[
            jax.ShapeDtypeStruct((NB, T, K), jnp.float32),
            jax.ShapeDtypeStruct((NB, T, K), jnp.int32),
            jax.ShapeDtypeStruct((NB, 1, T), jnp.float32),
        ],
    )(blk_flags, p2)

    vals = vals.reshape(N, K)
    idx = idx.reshape(N, K)
    ent = ent.reshape(N)

    w = vals / (jnp.sum(vals, axis=1, keepdims=True) + 1e-10)
    r_s = jnp.clip(omega_s, 0.0, 1.0)
    r_a = jax.nn.softplus(omega_a)
    r_b = jax.nn.softplus(omega_b)
    lam = r_s * jax.nn.sigmoid(r_a * (r_b - ent))
    is_mask = (xt == MASK_ID).astype(jnp.float32)

    s_idx = jnp.concatenate([xt[:, None], idx], axis=1).reshape(-1)
    s_par = jnp.concatenate(
        [is_mask[:, None], lam[:, None], w], axis=1).reshape(-1)

    emb3 = embedding_weight.astype(jnp.float32).reshape(V, 1, H)

    def row_spec(fn):
        return pl.BlockSpec((1, 1, H), fn)

    in_specs = [
        row_spec(lambda t, si, sp: (si[t * (1 + K)], 0, 0)),
        row_spec(lambda t, si, sp: (MASK_ID, 0, 0)),
    ]
    for k in range(K):
        in_specs.append(
            row_spec(functools.partial(
                lambda t, si, sp, kk: (si[t * (1 + K) + 1 + kk], 0, 0), kk=k))
        )

    out = pl.pallas_call(
        _combine_kernel,
        grid_spec=pltpu.PrefetchScalarGridSpec(
            num_scalar_prefetch=2,
            grid=(N,),
            in_specs=in_specs,
            out_specs=pl.BlockSpec((1, 1, H), lambda t, si, sp: (t, 0, 0)),
        ),
        out_shape=jax.ShapeDtypeStruct((N, 1, H), jnp.float32),
    )(s_idx, s_par, *([emb3] * (2 + K)))

    return out.reshape(B, S, H)


# speedup vs baseline: 142.9543x; 1.1073x over previous
import functools

import jax
import jax.numpy as jnp
from jax.experimental import pallas as pl
from jax.experimental.pallas import tpu as pltpu

MASK_ID = 0
K = 8
T = 8


def _stats_kernel(flag_ref, p_hbm, vals_ref, idx_ref, ent_ref, p_vmem, sem, *, V):
    i = pl.program_id(0)

    @pl.when(flag_ref[i] != 0)
    def _compute():
        cp = pltpu.make_async_copy(
            p_hbm.at[pl.ds(i * T, T), :], p_vmem, sem)
        cp.start()
        cp.wait()
        p = p_vmem[...]
        col = jax.lax.broadcasted_iota(jnp.int32, (T, V), 1)
        safe = jnp.where(p > 0.0, p, 1.0)
        ent = jnp.sum(jnp.where(p > 0.0, -p * jnp.log(safe), 0.0), axis=1)
        ent_ref[0, 0, :] = ent
        pz = jnp.where(col == MASK_ID, -1.0, p)
        vals = []
        idxs = []
        for _ in range(K):
            m = jnp.max(pz, axis=1)
            am = jnp.min(jnp.where(pz == m[:, None], col, V), axis=1)
            vals.append(m)
            idxs.append(am)
            pz = jnp.where(col == am[:, None], -1.0, pz)
        vals_ref[0, :, :] = jnp.stack(vals, axis=1)
        idx_ref[0, :, :] = jnp.stack(idxs, axis=1)

    @pl.when(flag_ref[i] == 0)
    def _skip():
        vals_ref[...] = jnp.zeros_like(vals_ref)
        idx_ref[...] = jnp.zeros_like(idx_ref)
        ent_ref[...] = jnp.zeros_like(ent_ref)


def _combine_kernel(idx_sref, par_sref, real_ref, mask_ref, *rest):
    e_refs = rest[:K]
    out_ref = rest[K]
    t = pl.program_id(0)
    lam = par_sref[t * (2 + K) + 1]
    is_mask = par_sref[t * (2 + K)] > 0.5
    acc = par_sref[t * (2 + K) + 2] * e_refs[0][0, 0, :]
    for k in range(1, K):
        acc = acc + par_sref[t * (2 + K) + 2 + k] * e_refs[k][0, 0, :]
    soft = (1.0 - lam) * mask_ref[0, 0, :] + lam * acc
    out_ref[0, 0, :] = jnp.where(is_mask, soft, real_ref[0, 0, :])


def kernel(x_t, probs, embedding_weight, omega_s, omega_a, omega_b):
    B, S, V = probs.shape
    H = embedding_weight.shape[1]
    N = B * S
    NB = N // T
    p2 = probs.reshape(N, V).astype(jnp.float32)
    xt = x_t.reshape(N).astype(jnp.int32)

    blk_flags = jnp.any(xt.reshape(NB, T) == MASK_ID, axis=1).astype(jnp.int32)

    vals, idx, ent = pl.pallas_call(
        functools.partial(_stats_kernel, V=V),
        grid_spec=pltpu.PrefetchScalarGridSpec(
            num_scalar_prefetch=1,
            grid=(NB,),
            in_specs=[pl.BlockSpec(memory_space=pl.ANY)],
            out_specs=[
                pl.BlockSpec((1, T, K), lambda i, f: (i, 0, 0)),
                pl.BlockSpec((1, T, K), lambda i, f: (i, 0, 0)),
                pl.BlockSpec((1, 1, T), lambda i, f: (i, 0, 0)),
            ],
            scratch_shapes=[
                pltpu.VMEM((T, V), jnp.float32),
                pltpu.SemaphoreType.DMA,
            ],
        ),
        out_shape=[
            jax.ShapeDtypeStruct((NB, T, K), jnp.float32),
            jax.ShapeDtypeStruct((NB, T, K), jnp.int32),
            jax.ShapeDtypeStruct((NB, 1, T), jnp.float32),
        ],
    )(blk_flags, p2)

    vals = vals.reshape(N, K)
    idx = idx.reshape(N, K)
    ent = ent.reshape(N)

    w = vals / (jnp.sum(vals, axis=1, keepdims=True) + 1e-10)
    r_s = jnp.clip(omega_s, 0.0, 1.0)
    r_a = jax.nn.softplus(omega_a)
    r_b = jax.nn.softplus(omega_b)
    lam = r_s * jax.nn.sigmoid(r_a * (r_b - ent))
    is_mask = (xt == MASK_ID).astype(jnp.float32)

    s_idx = jnp.concatenate([xt[:, None], idx], axis=1).reshape(-1)
    s_par = jnp.concatenate(
        [is_mask[:, None], lam[:, None], w], axis=1).reshape(-1)

    emb3 = embedding_weight.astype(jnp.float32).reshape(V, 1, H)

    def row_spec(fn):
        return pl.BlockSpec((1, 1, H), fn)

    in_specs = [
        row_spec(lambda t, si, sp: (si[t * (1 + K)], 0, 0)),
        row_spec(lambda t, si, sp: (MASK_ID, 0, 0)),
    ]
    for k in range(K):
        in_specs.append(
            row_spec(functools.partial(
                lambda t, si, sp, kk: (si[t * (1 + K) + 1 + kk], 0, 0), kk=k))
        )

    out = pl.pallas_call(
        _combine_kernel,
        grid_spec=pltpu.PrefetchScalarGridSpec(
            num_scalar_prefetch=2,
            grid=(N,),
            in_specs=in_specs,
            out_specs=pl.BlockSpec((1, 1, H), lambda t, si, sp: (t, 0, 0)),
        ),
        out_shape=jax.ShapeDtypeStruct((N, 1, H), jnp.float32),
    )(s_idx, s_par, *([emb3] * (2 + K)))

    return out.reshape(B, S, H)


# speedup vs baseline: 168.1033x; 1.1759x over previous
import functools

import jax
import jax.numpy as jnp
from jax import lax
from jax.experimental import pallas as pl
from jax.experimental.pallas import tpu as pltpu
from jax.experimental.pallas import tpu_sc as plsc

MASK_ID = 0
K = 8
T = 8
NW = 32
LANES = 16
NCOEF = 10


def _stats_kernel(flag_ref, p_hbm, vals_ref, idx_ref, ent_ref, p_vmem, sem, *, V):
    i = pl.program_id(0)

    @pl.when(flag_ref[i] != 0)
    def _compute():
        cp = pltpu.make_async_copy(
            p_hbm.at[pl.ds(i * T, T), :], p_vmem, sem)
        cp.start()
        cp.wait()
        p = p_vmem[...]
        col = jax.lax.broadcasted_iota(jnp.int32, (T, V), 1)
        safe = jnp.where(p > 0.0, p, 1.0)
        ent = jnp.sum(jnp.where(p > 0.0, -p * jnp.log(safe), 0.0), axis=1)
        ent_ref[0, 0, :] = ent
        pz = jnp.where(col == MASK_ID, -1.0, p)
        vals = []
        idxs = []
        for _ in range(K):
            m = jnp.max(pz, axis=1)
            am = jnp.min(jnp.where(pz == m[:, None], col, V), axis=1)
            vals.append(m)
            idxs.append(am)
            pz = jnp.where(col == am[:, None], -1.0, pz)
        vals_ref[0, :, :] = jnp.stack(vals, axis=1)
        idx_ref[0, :, :] = jnp.stack(idxs, axis=1)

    @pl.when(flag_ref[i] == 0)
    def _skip():
        vals_ref[...] = jnp.zeros_like(vals_ref)
        idx_ref[...] = jnp.zeros_like(idx_ref)
        ent_ref[...] = jnp.zeros_like(ent_ref)


def _sc_combine(t_per_w, n_chunk, idx_minor, H, table_hbm, idx_hbm, coef_hbm,
                out_hbm, idx_v, rows_v, mask_v, coef_v, out_v, sem):
    wid = lax.axis_index("s") * 2 + lax.axis_index("c")
    pltpu.sync_copy(idx_hbm.at[wid], idx_v)
    pltpu.sync_copy(coef_hbm.at[wid], coef_v)
    pltpu.sync_copy(table_hbm.at[MASK_ID], mask_v)
    cps = []
    for c in range(n_chunk):
        cps.append(pltpu.async_copy(
            table_hbm.at[idx_v.at[c]],
            rows_v.at[pl.ds(c * idx_minor, idx_minor)], sem))
    for cp in cps:
        cp.wait()

    nch = H // LANES

    def token_body(j, carry):
        cbase = j * (NCOEF * LANES)
        rbase = j * (1 + K)
        for c in range(nch):
            sl = pl.ds(c * LANES, LANES)
            acc = coef_v[pl.ds(cbase, LANES)] * mask_v[sl]
            for k in range(1 + K):
                cf = coef_v[pl.ds(cbase + (1 + k) * LANES, LANES)]
                acc = acc + cf * rows_v[rbase + k, sl]
            out_v[j, sl] = acc
        return carry

    lax.fori_loop(0, t_per_w, token_body, 0)
    pltpu.sync_copy(out_v, out_hbm.at[pl.ds(wid * t_per_w, t_per_w)])


def kernel(x_t, probs, embedding_weight, omega_s, omega_a, omega_b):
    B, S, V = probs.shape
    H = embedding_weight.shape[1]
    N = B * S
    NB = N // T
    p2 = probs.reshape(N, V).astype(jnp.float32)
    xt = x_t.reshape(N).astype(jnp.int32)

    blk_flags = jnp.any(xt.reshape(NB, T) == MASK_ID, axis=1).astype(jnp.int32)

    vals, idx, ent = pl.pallas_call(
        functools.partial(_stats_kernel, V=V),
        grid_spec=pltpu.PrefetchScalarGridSpec(
            num_scalar_prefetch=1,
            grid=(NB,),
            in_specs=[pl.BlockSpec(memory_space=pl.ANY)],
            out_specs=[
                pl.BlockSpec((1, T, K), lambda i, f: (i, 0, 0)),
                pl.BlockSpec((1, T, K), lambda i, f: (i, 0, 0)),
                pl.BlockSpec((1, 1, T), lambda i, f: (i, 0, 0)),
            ],
            scratch_shapes=[
                pltpu.VMEM((T, V), jnp.float32),
                pltpu.SemaphoreType.DMA,
            ],
        ),
        out_shape=[
            jax.ShapeDtypeStruct((NB, T, K), jnp.float32),
            jax.ShapeDtypeStruct((NB, T, K), jnp.int32),
            jax.ShapeDtypeStruct((NB, 1, T), jnp.float32),
        ],
    )(blk_flags, p2)

    vals = vals.reshape(N, K)
    idx = idx.reshape(N, K)
    ent = ent.reshape(N)

    w = vals / (jnp.sum(vals, axis=1, keepdims=True) + 1e-10)
    r_s = jnp.clip(omega_s, 0.0, 1.0)
    r_a = jax.nn.softplus(omega_a)
    r_b = jax.nn.softplus(omega_b)
    lam = r_s * jax.nn.sigmoid(r_a * (r_b - ent))
    is_mask = (xt == MASK_ID).astype(jnp.float32)

    c_mask = is_mask * (1.0 - lam)
    c_real = 1.0 - is_mask
    c_topk = (is_mask * lam)[:, None] * w
    coef = jnp.concatenate([c_mask[:, None], c_real[:, None], c_topk], axis=1)
    t_per_w = N // NW
    coef16 = jnp.broadcast_to(
        coef[:, :, None], (N, NCOEF, LANES)).reshape(NW, t_per_w * NCOEF * LANES)

    rows_per_w = t_per_w * (1 + K)
    idx_minor = 96
    n_chunk = rows_per_w // idx_minor
    s_idx = jnp.concatenate([xt[:, None], idx], axis=1)
    idx_arr = s_idx.reshape(NW, n_chunk, idx_minor)

    table = embedding_weight.astype(jnp.float32)

    sc_combine = functools.partial(
        pl.kernel,
        out_type=jax.ShapeDtypeStruct((N, H), jnp.float32),
        mesh=plsc.VectorSubcoreMesh(core_axis_name="c", subcore_axis_name="s"),
        scratch_types=[
            pltpu.VMEM((n_chunk, idx_minor), jnp.int32),
            pltpu.VMEM((rows_per_w, H), jnp.float32),
            pltpu.VMEM((H,), jnp.float32),
            pltpu.VMEM((t_per_w * NCOEF * LANES,), jnp.float32),
            pltpu.VMEM((t_per_w, H), jnp.float32),
            pltpu.SemaphoreType.DMA,
        ],
    )(functools.partial(_sc_combine, t_per_w, n_chunk, idx_minor, H))

    out = sc_combine(table, idx_arr, coef16)
    return out.reshape(B, S, H)


# speedup vs baseline: 340.1891x; 2.0237x over previous
import functools

import jax
import jax.numpy as jnp
from jax import lax
from jax.experimental import pallas as pl
from jax.experimental.pallas import tpu as pltpu
from jax.experimental.pallas import tpu_sc as plsc

MASK_ID = 0
K = 8
T = 8
NW = 32
LANES = 16
NCOEF = 10


def _stats_kernel(flag_ref, p_hbm, vals_ref, idx_ref, ent_ref, p_vmem, sem, *, V):
    i = pl.program_id(0)

    @pl.when(flag_ref[i] != 0)
    def _compute():
        cp = pltpu.make_async_copy(
            p_hbm.at[pl.ds(i * T, T), :], p_vmem, sem)
        cp.start()
        cp.wait()
        p = p_vmem[...]
        col = jax.lax.broadcasted_iota(jnp.int32, (T, V), 1)
        safe = jnp.where(p > 0.0, p, 1.0)
        ent = jnp.sum(jnp.where(p > 0.0, -p * jnp.log(safe), 0.0), axis=1)
        ent_ref[0, 0, :] = ent
        pz = jnp.where(col == MASK_ID, -1.0, p)
        vals = []
        idxs = []
        for _ in range(K):
            m = jnp.max(pz, axis=1)
            am = jnp.min(jnp.where(pz == m[:, None], col, V), axis=1)
            vals.append(m)
            idxs.append(am)
            pz = jnp.where(col == am[:, None], -1.0, pz)
        vals_ref[0, :, :] = jnp.stack(vals, axis=1)
        idx_ref[0, :, :] = jnp.stack(idxs, axis=1)

    @pl.when(flag_ref[i] == 0)
    def _skip():
        vals_ref[...] = jnp.zeros_like(vals_ref)
        slot = (jax.lax.broadcasted_iota(jnp.int32, (1, T, K), 1) * K
                + jax.lax.broadcasted_iota(jnp.int32, (1, T, K), 2))
        idx_ref[...] = i * (T * K) + slot
        ent_ref[...] = jnp.zeros_like(ent_ref)


def _sc_combine(t_per_w, n_chunk, idx_minor, H, table_hbm, idx_hbm, coef_hbm,
                out_hbm, idx_v, rows_v, mask_v, coef_v, out_v, sem):
    wid = lax.axis_index("s") * 2 + lax.axis_index("c")
    pltpu.sync_copy(idx_hbm.at[wid], idx_v)
    pltpu.sync_copy(coef_hbm.at[wid], coef_v)
    pltpu.sync_copy(table_hbm.at[MASK_ID], mask_v)
    cps = []
    for c in range(n_chunk):
        cps.append(pltpu.async_copy(
            table_hbm.at[idx_v.at[c]],
            rows_v.at[pl.ds(c * idx_minor, idx_minor)], sem))
    for cp in cps:
        cp.wait()

    nch = H // LANES

    def token_body(j, carry):
        cbase = j * (NCOEF * LANES)
        rbase = j * (1 + K)
        for c in range(nch):
            sl = pl.ds(c * LANES, LANES)
            acc = coef_v[pl.ds(cbase, LANES)] * mask_v[sl]
            for k in range(1 + K):
                cf = coef_v[pl.ds(cbase + (1 + k) * LANES, LANES)]
                acc = acc + cf * rows_v[rbase + k, sl]
            out_v[j, sl] = acc
        return carry

    lax.fori_loop(0, t_per_w, token_body, 0)
    pltpu.sync_copy(out_v, out_hbm.at[pl.ds(wid * t_per_w, t_per_w)])


def kernel(x_t, probs, embedding_weight, omega_s, omega_a, omega_b):
    B, S, V = probs.shape
    H = embedding_weight.shape[1]
    N = B * S
    NB = N // T
    p2 = probs.reshape(N, V).astype(jnp.float32)
    xt = x_t.reshape(N).astype(jnp.int32)

    blk_flags = jnp.any(xt.reshape(NB, T) == MASK_ID, axis=1).astype(jnp.int32)

    vals, idx, ent = pl.pallas_call(
        functools.partial(_stats_kernel, V=V),
        grid_spec=pltpu.PrefetchScalarGridSpec(
            num_scalar_prefetch=1,
            grid=(NB,),
            in_specs=[pl.BlockSpec(memory_space=pl.ANY)],
            out_specs=[
                pl.BlockSpec((1, T, K), lambda i, f: (i, 0, 0)),
                pl.BlockSpec((1, T, K), lambda i, f: (i, 0, 0)),
                pl.BlockSpec((1, 1, T), lambda i, f: (i, 0, 0)),
            ],
            scratch_shapes=[
                pltpu.VMEM((T, V), jnp.float32),
                pltpu.SemaphoreType.DMA,
            ],
        ),
        out_shape=[
            jax.ShapeDtypeStruct((NB, T, K), jnp.float32),
            jax.ShapeDtypeStruct((NB, T, K), jnp.int32),
            jax.ShapeDtypeStruct((NB, 1, T), jnp.float32),
        ],
    )(blk_flags, p2)

    vals = vals.reshape(N, K)
    idx = idx.reshape(N, K)
    ent = ent.reshape(N)

    w = vals / (jnp.sum(vals, axis=1, keepdims=True) + 1e-10)
    r_s = jnp.clip(omega_s, 0.0, 1.0)
    r_a = jax.nn.softplus(omega_a)
    r_b = jax.nn.softplus(omega_b)
    lam = r_s * jax.nn.sigmoid(r_a * (r_b - ent))
    is_mask = (xt == MASK_ID).astype(jnp.float32)

    c_mask = is_mask * (1.0 - lam)
    c_real = 1.0 - is_mask
    c_topk = (is_mask * lam)[:, None] * w
    coef = jnp.concatenate([c_mask[:, None], c_real[:, None], c_topk], axis=1)
    t_per_w = N // NW
    coef16 = jnp.broadcast_to(
        coef[:, :, None], (N, NCOEF, LANES)).reshape(NW, t_per_w * NCOEF * LANES)

    rows_per_w = t_per_w * (1 + K)
    idx_minor = 96
    n_chunk = rows_per_w // idx_minor
    s_idx = jnp.concatenate([xt[:, None], idx], axis=1)
    idx_arr = s_idx.reshape(NW, n_chunk, idx_minor)

    table = embedding_weight.astype(jnp.float32)

    sc_combine = functools.partial(
        pl.kernel,
        out_type=jax.ShapeDtypeStruct((N, H), jnp.float32),
        mesh=plsc.VectorSubcoreMesh(core_axis_name="c", subcore_axis_name="s"),
        scratch_types=[
            pltpu.VMEM((n_chunk, idx_minor), jnp.int32),
            pltpu.VMEM((rows_per_w, H), jnp.float32),
            pltpu.VMEM((H,), jnp.float32),
            pltpu.VMEM((t_per_w * NCOEF * LANES,), jnp.float32),
            pltpu.VMEM((t_per_w, H), jnp.float32),
            pltpu.SemaphoreType.DMA,
        ],
    )(functools.partial(_sc_combine, t_per_w, n_chunk, idx_minor, H))

    out = sc_combine(table, idx_arr, coef16)
    return out.reshape(B, S, H)


# speedup vs baseline: 939.0375x; 2.7603x over previous
import functools

import jax
import jax.numpy as jnp
from jax import lax
from jax.experimental import pallas as pl
from jax.experimental.pallas import tpu as pltpu
from jax.experimental.pallas import tpu_sc as plsc

MASK_ID = 0
K = 8
T = 8
NW = 32
LANES = 16
NCOEF = 10


def _stats_kernel(cnt_ref, ids_ref, p_hbm, vals_ref, idx_ref, ent_ref,
                  p_vmem, sem, *, V, NBC):
    cnt = cnt_ref[0]

    def block_body(b, carry):
        @pl.when(b * T < cnt)
        def _compute():
            cps = []
            for j in range(T):
                tok = ids_ref[b * T + j]
                cps.append(pltpu.make_async_copy(
                    p_hbm.at[pl.ds(tok, 1), :],
                    p_vmem.at[pl.ds(j, 1), :], sem))
            for cp in cps:
                cp.start()
            for cp in cps:
                cp.wait()
            p = p_vmem[...]
            col = jax.lax.broadcasted_iota(jnp.int32, (T, V), 1)
            safe = jnp.where(p > 0.0, p, 1.0)
            ent = jnp.sum(jnp.where(p > 0.0, -p * jnp.log(safe), 0.0), axis=1)
            ent_ref[pl.ds(b, 1)] = ent[None, None, :]
            pz = jnp.where(col == MASK_ID, -1.0, p)
            vals = []
            idxs = []
            for _ in range(K):
                m = jnp.max(pz, axis=1)
                am = jnp.min(jnp.where(pz == m[:, None], col, V), axis=1)
                vals.append(m)
                idxs.append(am)
                pz = jnp.where(col == am[:, None], -1.0, pz)
            vals_ref[pl.ds(b, 1)] = jnp.stack(vals, axis=1)[None]
            idx_ref[pl.ds(b, 1)] = jnp.stack(idxs, axis=1)[None]

        return carry

    lax.fori_loop(0, NBC, block_body, 0)


def _sc_combine(t_per_w, n_chunk, idx_minor, H, table_hbm, idx_hbm, coef_hbm,
                out_hbm, idx_v, rows_v, mask_v, coef_v, out_v, sem):
    wid = lax.axis_index("s") * 2 + lax.axis_index("c")
    pltpu.sync_copy(idx_hbm.at[wid], idx_v)
    pltpu.sync_copy(coef_hbm.at[wid], coef_v)
    pltpu.sync_copy(table_hbm.at[MASK_ID], mask_v)
    cps = []
    for c in range(n_chunk):
        cps.append(pltpu.async_copy(
            table_hbm.at[idx_v.at[c]],
            rows_v.at[pl.ds(c * idx_minor, idx_minor)], sem))
    for cp in cps:
        cp.wait()

    nch = H // LANES

    def token_body(j, carry):
        cbase = j * (NCOEF * LANES)
        rbase = j * (1 + K)
        for c in range(nch):
            sl = pl.ds(c * LANES, LANES)
            acc = coef_v[pl.ds(cbase, LANES)] * mask_v[sl]
            for k in range(1 + K):
                cf = coef_v[pl.ds(cbase + (1 + k) * LANES, LANES)]
                acc = acc + cf * rows_v[rbase + k, sl]
            out_v[j, sl] = acc
        return carry

    lax.fori_loop(0, t_per_w, token_body, 0)
    pltpu.sync_copy(out_v, out_hbm.at[pl.ds(wid * t_per_w, t_per_w)])


def kernel(x_t, probs, embedding_weight, omega_s, omega_a, omega_b):
    B, S, V = probs.shape
    H = embedding_weight.shape[1]
    N = B * S
    NB = N // T
    p2 = probs.reshape(N, V).astype(jnp.float32)
    xt = x_t.reshape(N).astype(jnp.int32)

    is_mask_b = xt == MASK_ID
    cnt = jnp.sum(is_mask_b.astype(jnp.int32)).reshape(1)
    comp_ids = jnp.argsort(jnp.where(is_mask_b, 0, 1), stable=True)
    comp_ids = comp_ids.astype(jnp.int32)

    vals_c, idx_c, ent_c = pl.pallas_call(
        functools.partial(_stats_kernel, V=V, NBC=NB),
        grid_spec=pltpu.PrefetchScalarGridSpec(
            num_scalar_prefetch=2,
            grid=(1,),
            in_specs=[pl.BlockSpec(memory_space=pl.ANY)],
            out_specs=[
                pl.BlockSpec((NB, T, K), lambda i, c, d: (0, 0, 0)),
                pl.BlockSpec((NB, T, K), lambda i, c, d: (0, 0, 0)),
                pl.BlockSpec((NB, 1, T), lambda i, c, d: (0, 0, 0)),
            ],
            scratch_shapes=[
                pltpu.VMEM((T, V), jnp.float32),
                pltpu.SemaphoreType.DMA,
            ],
        ),
        out_shape=[
            jax.ShapeDtypeStruct((NB, T, K), jnp.float32),
            jax.ShapeDtypeStruct((NB, T, K), jnp.int32),
            jax.ShapeDtypeStruct((NB, 1, T), jnp.float32),
        ],
    )(cnt, comp_ids, p2)

    vals = jnp.zeros((N, K), jnp.float32).at[comp_ids].set(vals_c.reshape(N, K))
    idx = jnp.zeros((N, K), jnp.int32).at[comp_ids].set(idx_c.reshape(N, K))
    ent = jnp.zeros((N,), jnp.float32).at[comp_ids].set(ent_c.reshape(N))

    w = vals / (jnp.sum(vals, axis=1, keepdims=True) + 1e-10)
    r_s = jnp.clip(omega_s, 0.0, 1.0)
    r_a = jax.nn.softplus(omega_a)
    r_b = jax.nn.softplus(omega_b)
    lam = r_s * jax.nn.sigmoid(r_a * (r_b - ent))
    is_mask = is_mask_b.astype(jnp.float32)

    spread = (jnp.arange(N * K, dtype=jnp.int32).reshape(N, K)) % V
    idx = jnp.where(is_mask_b[:, None], idx, spread)

    c_mask = jnp.where(is_mask_b, 1.0 - lam, 0.0)
    c_real = 1.0 - is_mask
    c_topk = jnp.where(is_mask_b[:, None], lam[:, None] * w, 0.0)
    coef = jnp.concatenate([c_mask[:, None], c_real[:, None], c_topk], axis=1)
    t_per_w = N // NW
    coef16 = jnp.broadcast_to(
        coef[:, :, None], (N, NCOEF, LANES)).reshape(NW, t_per_w * NCOEF * LANES)

    rows_per_w = t_per_w * (1 + K)
    idx_minor = 96
    n_chunk = rows_per_w // idx_minor
    s_idx = jnp.concatenate([xt[:, None], idx], axis=1)
    idx_arr = s_idx.reshape(NW, n_chunk, idx_minor)

    table = embedding_weight.astype(jnp.float32)

    sc_combine = functools.partial(
        pl.kernel,
        out_type=jax.ShapeDtypeStruct((N, H), jnp.float32),
        mesh=plsc.VectorSubcoreMesh(core_axis_name="c", subcore_axis_name="s"),
        scratch_types=[
            pltpu.VMEM((n_chunk, idx_minor), jnp.int32),
            pltpu.VMEM((rows_per_w, H), jnp.float32),
            pltpu.VMEM((H,), jnp.float32),
            pltpu.VMEM((t_per_w * NCOEF * LANES,), jnp.float32),
            pltpu.VMEM((t_per_w, H), jnp.float32),
            pltpu.SemaphoreType.DMA,
        ],
    )(functools.partial(_sc_combine, t_per_w, n_chunk, idx_minor, H))

    out = sc_combine(table, idx_arr, coef16)
    return out.reshape(B, S, H)


# speedup vs baseline: 1028.4330x; 1.0952x over previous
import functools

import jax
import jax.numpy as jnp
from jax import lax
from jax.experimental import pallas as pl
from jax.experimental.pallas import tpu as pltpu
from jax.experimental.pallas import tpu_sc as plsc

MASK_ID = 0
K = 8
T = 8
NW = 32
LANES = 16
NCOEF = 10


def _stats_kernel(cnt_ref, ids_ref, omg_ref, p_hbm, idx_init_ref,
                  coef_init_ref, idx_ref, coef_ref, p_vmem, sem, *, V, NBC):
    cnt = cnt_ref[0]
    r_s = omg_ref[0]
    r_a = omg_ref[1]
    r_b = omg_ref[2]
    idx_ref[...] = idx_init_ref[...]
    coef_ref[...] = coef_init_ref[...]

    def block_body(b, carry):
        @pl.when(b * T < cnt)
        def _compute():
            toks = [ids_ref[b * T + j] for j in range(T)]
            cps = []
            for j in range(T):
                cps.append(pltpu.make_async_copy(
                    p_hbm.at[pl.ds(toks[j], 1), :],
                    p_vmem.at[pl.ds(j, 1), :], sem))
            for cp in cps:
                cp.start()
            for cp in cps:
                cp.wait()
            p = p_vmem[...]
            col = jax.lax.broadcasted_iota(jnp.int32, (T, V), 1)
            safe = jnp.where(p > 0.0, p, 1.0)
            ent = jnp.sum(jnp.where(p > 0.0, -p * jnp.log(safe), 0.0), axis=1)
            pz = jnp.where(col == MASK_ID, -1.0, p)
            vals = []
            idxs = []
            for _ in range(K):
                m = jnp.max(pz, axis=1)
                am = jnp.min(jnp.where(pz == m[:, None], col, V), axis=1)
                vals.append(m)
                idxs.append(am)
                pz = jnp.where(col == am[:, None], -1.0, pz)
            vals = jnp.stack(vals, axis=1)
            idxs = jnp.stack(idxs, axis=1)
            w = vals / (jnp.sum(vals, axis=1, keepdims=True) + 1e-10)
            lam = r_s * jax.nn.sigmoid(r_a * (r_b - ent))
            crow = jnp.concatenate(
                [(1.0 - lam)[:, None], jnp.zeros((T, 1), jnp.float32),
                 lam[:, None] * w], axis=1)
            coef_blk = jnp.broadcast_to(
                crow[:, :, None], (T, NCOEF, LANES))
            tokcol = jnp.stack(toks, axis=0).astype(jnp.int32)[:, None]
            irow = jnp.concatenate([tokcol, idxs], axis=1)
            for j in range(T):
                @pl.when(b * T + j < cnt)
                def _store(j=j):
                    idx_ref[pl.ds(toks[j], 1)] = irow[j][None]
                    coef_ref[pl.ds(toks[j], 1)] = coef_blk[j][None]

        return carry

    lax.fori_loop(0, NBC, block_body, 0)


def _sc_combine(t_per_w, n_chunk, idx_minor, H, table_hbm, idx_hbm, coef_hbm,
                out_hbm, idx_v, rows_v, mask_v, coef_v, out_v, sem):
    wid = lax.axis_index("s") * 2 + lax.axis_index("c")
    pltpu.sync_copy(idx_hbm.at[wid], idx_v)
    pltpu.sync_copy(coef_hbm.at[wid], coef_v)
    pltpu.sync_copy(table_hbm.at[MASK_ID], mask_v)
    cps = []
    for c in range(n_chunk):
        cps.append(pltpu.async_copy(
            table_hbm.at[idx_v.at[c]],
            rows_v.at[pl.ds(c * idx_minor, idx_minor)], sem))
    for cp in cps:
        cp.wait()

    nch = H // LANES

    def token_body(j, carry):
        cbase = j * (NCOEF * LANES)
        rbase = j * (1 + K)
        for c in range(nch):
            sl = pl.ds(c * LANES, LANES)
            acc = coef_v[pl.ds(cbase, LANES)] * mask_v[sl]
            for k in range(1 + K):
                cf = coef_v[pl.ds(cbase + (1 + k) * LANES, LANES)]
                acc = acc + cf * rows_v[rbase + k, sl]
            out_v[j, sl] = acc
        return carry

    lax.fori_loop(0, t_per_w, token_body, 0)
    pltpu.sync_copy(out_v, out_hbm.at[pl.ds(wid * t_per_w, t_per_w)])


def kernel(x_t, probs, embedding_weight, omega_s, omega_a, omega_b):
    B, S, V = probs.shape
    H = embedding_weight.shape[1]
    N = B * S
    NB = N // T
    p2 = probs.reshape(N, V).astype(jnp.float32)
    xt = x_t.reshape(N).astype(jnp.int32)

    is_mask_b = xt == MASK_ID
    cnt = jnp.sum(is_mask_b.astype(jnp.int32)).reshape(1)
    comp_ids = jnp.argsort(jnp.where(is_mask_b, 0, 1), stable=True)
    comp_ids = comp_ids.astype(jnp.int32)
    omg = jnp.stack([
        jnp.clip(omega_s, 0.0, 1.0).astype(jnp.float32),
        jax.nn.softplus(omega_a).astype(jnp.float32),
        jax.nn.softplus(omega_b).astype(jnp.float32),
    ])

    spread = (jnp.arange(N * K, dtype=jnp.int32) % (V - 1) + 1).reshape(N, K)
    idx_init = jnp.concatenate([xt[:, None], spread], axis=1)
    coef_init = jnp.broadcast_to(
        (jnp.arange(NCOEF) == 1).astype(jnp.float32)[None, :, None],
        (N, NCOEF, LANES))

    idx, coef16 = pl.pallas_call(
        functools.partial(_stats_kernel, V=V, NBC=NB),
        grid_spec=pltpu.PrefetchScalarGridSpec(
            num_scalar_prefetch=3,
            grid=(1,),
            in_specs=[
                pl.BlockSpec(memory_space=pl.ANY),
                pl.BlockSpec((N, 1 + K), lambda i, c, d, o: (0, 0)),
                pl.BlockSpec((N, NCOEF, LANES), lambda i, c, d, o: (0, 0, 0)),
            ],
            out_specs=[
                pl.BlockSpec((N, 1 + K), lambda i, c, d, o: (0, 0)),
                pl.BlockSpec((N, NCOEF, LANES), lambda i, c, d, o: (0, 0, 0)),
            ],
            scratch_shapes=[
                pltpu.VMEM((T, V), jnp.float32),
                pltpu.SemaphoreType.DMA,
            ],
        ),
        out_shape=[
            jax.ShapeDtypeStruct((N, 1 + K), jnp.int32),
            jax.ShapeDtypeStruct((N, NCOEF, LANES), jnp.float32),
        ],
    )(cnt, comp_ids, omg, p2, idx_init, coef_init)

    t_per_w = N // NW
    rows_per_w = t_per_w * (1 + K)
    idx_minor = 96
    n_chunk = rows_per_w // idx_minor
    idx_arr = idx.reshape(NW, n_chunk, idx_minor)
    coef16 = coef16.reshape(NW, t_per_w * NCOEF * LANES)

    table = embedding_weight.astype(jnp.float32)

    sc_combine = functools.partial(
        pl.kernel,
        out_type=jax.ShapeDtypeStruct((N, H), jnp.float32),
        mesh=plsc.VectorSubcoreMesh(core_axis_name="c", subcore_axis_name="s"),
        scratch_types=[
            pltpu.VMEM((n_chunk, idx_minor), jnp.int32),
            pltpu.VMEM((rows_per_w, H), jnp.float32),
            pltpu.VMEM((H,), jnp.float32),
            pltpu.VMEM((t_per_w * NCOEF * LANES,), jnp.float32),
            pltpu.VMEM((t_per_w, H), jnp.float32),
            pltpu.SemaphoreType.DMA,
        ],
    )(functools.partial(_sc_combine, t_per_w, n_chunk, idx_minor, H))

    out = sc_combine(table, idx_arr, coef16)
    return out.reshape(B, S, H)
